# baseline (speedup 1.0000x reference)
import functools

import jax
import jax.numpy as jnp
import numpy as np
from jax import lax
from jax.experimental import pallas as pl
from jax.experimental.pallas import tpu as pltpu
from jax.experimental.pallas import tpu_sc as plsc

PI = float(np.pi)
K = 32
CUTOFF = 10.0
NF = 64

_OFFSET_NP = np.linspace(0.0, CUTOFF, NF).astype(np.float32)
_COEFF = float(np.float32(-0.5) / np.float32((_OFFSET_NP[1] - _OFFSET_NP[0]) ** 2))
_LOG2 = float(np.log(2.0))


_BQ = 200


def _knn_body(pq_ref, pctxT_ref, nbrT_ref, d2T_ref, d2_scr, d2f_scr):
    nq, nc = d2_scr.shape
    pq = pq_ref[...]
    pT = pctxT_ref[...]
    qq = jnp.sum(pq * pq, axis=1, keepdims=True)
    cc = jnp.sum(pT * pT, axis=0, keepdims=True)
    ip_sel = lax.dot_general(
        pq.astype(jnp.bfloat16), pT.astype(jnp.bfloat16),
        (((1,), (0,)), ((), ())), preferred_element_type=jnp.float32)
    d2_scr[...] = qq + cc - 2.0 * ip_sel
    d0 = pq[:, 0:1] - pT[0:1, :]
    d1 = pq[:, 1:2] - pT[1:2, :]
    d2c = pq[:, 2:3] - pT[2:3, :]
    d2f_scr[...] = (d0 * d0 + d1 * d1) + d2c * d2c
    col = lax.broadcasted_iota(jnp.int32, (nq, nc), 1)

    def step(k, carry):
        d2v = d2_scr[...]
        m = jnp.min(d2v, axis=1)
        sel = jnp.where(d2v == m[:, None], col, jnp.int32(2**30))
        idx = jnp.min(sel, axis=1)
        hit = col == idx[:, None]
        mf = jnp.min(jnp.where(hit, d2f_scr[...], jnp.float32(jnp.inf)),
                     axis=1)
        d2_scr[...] = jnp.where(hit, jnp.float32(jnp.inf), d2v)
        nbrT_ref[0, pl.ds(k, 1), :] = idx[None, :]
        d2T_ref[0, pl.ds(k, 1), :] = mf[None, :]
        return carry

    lax.fori_loop(0, K, step, 0)


def _knn(pos_query, pos_ctx_T):
    nq = pos_query.shape[0]
    nc = pos_ctx_T.shape[1]
    grid = nq // _BQ
    return pl.pallas_call(
        _knn_body,
        grid=(grid,),
        in_specs=[
            pl.BlockSpec((_BQ, 3), lambda i: (i, 0)),
            pl.BlockSpec((3, nc), lambda i: (0, 0)),
        ],
        out_specs=[
            pl.BlockSpec((1, K, _BQ), lambda i: (i, 0, 0)),
            pl.BlockSpec((1, K, _BQ), lambda i: (i, 0, 0)),
        ],
        out_shape=[
            jax.ShapeDtypeStruct((grid, K, _BQ), jnp.int32),
            jax.ShapeDtypeStruct((grid, K, _BQ), jnp.float32),
        ],
        scratch_shapes=[pltpu.VMEM((_BQ, nc), jnp.float32),
                        pltpu.VMEM((_BQ, nc), jnp.float32)],
    )(pos_query, pos_ctx_T)


def _x1_body(na_ref, w_ref, out_ref):
    out_ref[...] = jnp.dot(na_ref[...], w_ref[...],
                           preferred_element_type=jnp.float32)


def _x1(node_attr_ctx, lin1_W):
    nc = node_attr_ctx.shape[0]
    return pl.pallas_call(
        _x1_body,
        out_shape=jax.ShapeDtypeStruct((nc, NF), jnp.float32),
    )(node_attr_ctx, lin1_W)


_SC_CHUNK = 80


def _gather(table, idx):
    e = idx.shape[0]
    info = plsc.get_sparse_core_info()
    ncores, nsub = info.num_cores, info.num_subcores
    nw = ncores * nsub
    b_per_w = e // nw
    nchunks = b_per_w // _SC_CHUNK
    mesh = plsc.VectorSubcoreMesh(core_axis_name="c", subcore_axis_name="s")

    @functools.partial(
        pl.kernel,
        out_type=jax.ShapeDtypeStruct((e, NF), jnp.float32),
        mesh=mesh,
        scratch_types=[
            pltpu.VMEM((_SC_CHUNK,), jnp.int32),
            pltpu.VMEM((_SC_CHUNK, NF), jnp.float32),
            pltpu.SemaphoreType.DMA,
        ],
        compiler_params=pltpu.CompilerParams(use_tc_tiling_on_sc=False),
    )
    def gather_kernel(table_hbm, idx_hbm, out_hbm, idx_v, rows_v, sem):
        wid = lax.axis_index("s") * ncores + lax.axis_index("c")
        base = wid * b_per_w

        def chunk(j, carry):
            off = base + j * _SC_CHUNK
            pltpu.sync_copy(idx_hbm.at[pl.ds(off, _SC_CHUNK)], idx_v)
            pltpu.async_copy(table_hbm.at[idx_v], rows_v, sem).wait()
            pltpu.sync_copy(rows_v, out_hbm.at[pl.ds(off, _SC_CHUNK)])
            return carry

        lax.fori_loop(0, nchunks, chunk, 0)

    return gather_kernel(table, idx)


_BQ2 = 200


def _combine_body(d2e_ref, g_ref, off_ref, dn1_ref, dn1b_ref, dn2_ref,
                  dn2b_ref, lin2_ref, lin2b_ref, c1_ref, c1b_ref, c2_ref,
                  c2b_ref, out_ref):
    d2b = jnp.maximum(d2e_ref[...], 0.0)
    dist = jnp.sqrt(d2b)
    delta = dist - off_ref[...]
    g = jnp.exp(_COEFF * delta * delta)
    t = jnp.dot(g, dn1_ref[...], preferred_element_type=jnp.float32)
    t = t + dn1b_ref[...]
    t = jnp.log(1.0 + jnp.exp(t)) - _LOG2
    w = jnp.dot(t, dn2_ref[...], preferred_element_type=jnp.float32)
    w = w + dn2b_ref[...]
    c = 0.5 * (jnp.cos(dist * (PI / CUTOFF)) + 1.0)
    c = jnp.where(dist <= CUTOFF, c, 0.0)
    p = (w * c) * g_ref[...]
    y0 = jnp.sum(p.reshape(_BQ2, K, NF), axis=1)
    csum = jnp.sum(c.reshape(_BQ2, K, 1), axis=1)
    y = jnp.dot(y0, lin2_ref[...], preferred_element_type=jnp.float32)
    y = y + csum * lin2b_ref[...]
    h = jnp.dot(y, c1_ref[...], preferred_element_type=jnp.float32)
    h = h + c1b_ref[...]
    h = jnp.log(1.0 + jnp.exp(h)) - _LOG2
    out_ref[...] = jnp.dot(h, c2_ref[...],
                           preferred_element_type=jnp.float32) + c2b_ref[...]


def _combine(d2e, g_rows, offset, dn1_W, dn1_b, dn2_W, dn2_b, lin2_W, lin2_b,
             cls1_W, cls1_b, cls2_W, cls2_b):
    e = d2e.shape[0]
    nq = e // K
    o = cls2_W.shape[1]
    grid = nq // _BQ2
    eb = _BQ2 * K
    full = lambda a: pl.BlockSpec(a.shape, lambda i: tuple(0 for _ in a.shape))
    return pl.pallas_call(
        _combine_body,
        grid=(grid,),
        in_specs=[
            pl.BlockSpec((eb, 1), lambda i: (i, 0)),
            pl.BlockSpec((eb, NF), lambda i: (i, 0)),
            full(offset), full(dn1_W), full(dn1_b), full(dn2_W), full(dn2_b),
            full(lin2_W), full(lin2_b), full(cls1_W), full(cls1_b),
            full(cls2_W), full(cls2_b),
        ],
        out_specs=pl.BlockSpec((_BQ2, o), lambda i: (i, 0)),
        out_shape=jax.ShapeDtypeStruct((nq, o), jnp.float32),
    )(d2e, g_rows, offset, dn1_W, dn1_b, dn2_W, dn2_b, lin2_W, lin2_b,
      cls1_W, cls1_b, cls2_W, cls2_b)


def kernel(pos_query, pos_ctx, node_attr_ctx, lin1_W, lin2_W, lin2_b,
           dn1_W, dn1_b, dn2_W, dn2_b, cls1_W, cls1_b, cls2_W, cls2_b):
    nq = pos_query.shape[0]
    pos_ctx_T = pos_ctx.T
    x1 = _x1(node_attr_ctx, lin1_W)
    nbr_t, d2_t = _knn(pos_query, pos_ctx_T)
    col = nbr_t.transpose(0, 2, 1).reshape(-1)
    d2e = d2_t.transpose(0, 2, 1).reshape(-1, 1)
    g_rows = _gather(x1, col)
    offset = jnp.asarray(_OFFSET_NP).reshape(1, NF)
    return _combine(
        d2e, g_rows, offset, dn1_W, dn1_b.reshape(1, NF), dn2_W,
        dn2_b.reshape(1, NF), lin2_W, lin2_b.reshape(1, NF), cls1_W,
        cls1_b.reshape(1, NF), cls2_W, cls2_b.reshape(1, -1))


# speedup vs baseline: 1.1860x; 1.1860x over previous
import functools

import jax
import jax.numpy as jnp
import numpy as np
from jax import lax
from jax.experimental import pallas as pl
from jax.experimental.pallas import tpu as pltpu
from jax.experimental.pallas import tpu_sc as plsc

PI = float(np.pi)
K = 32
CUTOFF = 10.0
NF = 64
NP3 = 16

_OFFSET_NP = np.linspace(0.0, CUTOFF, NF).astype(np.float32)
_COEFF = float(np.float32(-0.5) / np.float32((_OFFSET_NP[1] - _OFFSET_NP[0]) ** 2))
_LOG2 = float(np.log(2.0))


_BQ = 200


def _knn_body(pq_ref, pctxT_ref, nbrT_ref, d2_scr):
    nq, nc = d2_scr.shape
    pq = pq_ref[...]
    pT = pctxT_ref[...]
    qq = jnp.sum(pq * pq, axis=1, keepdims=True)
    cc = jnp.sum(pT * pT, axis=0, keepdims=True)
    ip_sel = lax.dot_general(
        pq.astype(jnp.bfloat16), pT.astype(jnp.bfloat16),
        (((1,), (0,)), ((), ())), preferred_element_type=jnp.float32)
    d2_scr[...] = qq + cc - 2.0 * ip_sel
    col = lax.broadcasted_iota(jnp.int32, (nq, nc), 1)
    m0 = jnp.min(d2_scr[...], axis=1)

    def step(k, m):
        d2v = d2_scr[...]
        sel = jnp.where(d2v == m[:, None], col, jnp.int32(2**30))
        idx = jnp.min(sel, axis=1)
        d2n = jnp.where(col == idx[:, None], jnp.float32(jnp.inf), d2v)
        d2_scr[...] = d2n
        nbrT_ref[0, pl.ds(k, 1), :] = idx[None, :]
        return jnp.min(d2n, axis=1)

    lax.fori_loop(0, K, step, m0)


def _knn(pos_query, pos_ctx_T):
    nq = pos_query.shape[0]
    nc = pos_ctx_T.shape[1]
    grid = nq // _BQ
    return pl.pallas_call(
        _knn_body,
        grid=(grid,),
        in_specs=[
            pl.BlockSpec((_BQ, 3), lambda i: (i, 0)),
            pl.BlockSpec((3, nc), lambda i: (0, 0)),
        ],
        out_specs=pl.BlockSpec((1, K, _BQ), lambda i: (i, 0, 0)),
        out_shape=jax.ShapeDtypeStruct((grid, K, _BQ), jnp.int32),
        scratch_shapes=[pltpu.VMEM((_BQ, nc), jnp.float32)],
    )(pos_query, pos_ctx_T)


def _x1_body(na_ref, w_ref, out_ref):
    out_ref[...] = jnp.dot(na_ref[...], w_ref[...],
                           preferred_element_type=jnp.float32)


def _x1(node_attr_ctx, lin1_W):
    nc = node_attr_ctx.shape[0]
    return pl.pallas_call(
        _x1_body,
        out_shape=jax.ShapeDtypeStruct((nc, NF), jnp.float32),
    )(node_attr_ctx, lin1_W)


_SC_CHUNK = 80


def _gather(table, pos_pad, idx):
    e = idx.shape[0]
    info = plsc.get_sparse_core_info()
    ncores, nsub = info.num_cores, info.num_subcores
    nw = ncores * nsub
    b_per_w = e // nw
    nchunks = b_per_w // _SC_CHUNK
    mesh = plsc.VectorSubcoreMesh(core_axis_name="c", subcore_axis_name="s")

    @functools.partial(
        pl.kernel,
        out_type=[jax.ShapeDtypeStruct((e, NF), jnp.float32),
                  jax.ShapeDtypeStruct((e, NP3), jnp.float32)],
        mesh=mesh,
        scratch_types=[
            pltpu.VMEM((_SC_CHUNK,), jnp.int32),
            pltpu.VMEM((_SC_CHUNK, NF), jnp.float32),
            pltpu.VMEM((_SC_CHUNK, NP3), jnp.float32),
            pltpu.SemaphoreType.DMA,
            pltpu.SemaphoreType.DMA,
        ],
        compiler_params=pltpu.CompilerParams(use_tc_tiling_on_sc=False),
    )
    def gather_kernel(table_hbm, pos_hbm, idx_hbm, out_hbm, outp_hbm,
                      idx_v, rows_v, pos_v, sem, semp):
        wid = lax.axis_index("s") * ncores + lax.axis_index("c")
        base = wid * b_per_w

        def chunk(j, carry):
            off = base + j * _SC_CHUNK
            pltpu.sync_copy(idx_hbm.at[pl.ds(off, _SC_CHUNK)], idx_v)
            cp1 = pltpu.async_copy(table_hbm.at[idx_v], rows_v, sem)
            cp2 = pltpu.async_copy(pos_hbm.at[idx_v], pos_v, semp)
            cp1.wait()
            cp2.wait()
            pltpu.sync_copy(rows_v, out_hbm.at[pl.ds(off, _SC_CHUNK)])
            pltpu.sync_copy(pos_v, outp_hbm.at[pl.ds(off, _SC_CHUNK)])
            return carry

        lax.fori_loop(0, nchunks, chunk, 0)

    return gather_kernel(table, pos_pad, idx)


_BQ2 = 200


def _combine_body(pq_ref, g_ref, pg_ref, off_ref, dn1_ref, dn1b_ref, dn2_ref,
                  dn2b_ref, lin2_ref, lin2b_ref, c1_ref, c1b_ref, c2_ref,
                  c2b_ref, out_ref):
    pqb = pq_ref[...]
    pg = pg_ref[...]
    eb = pg.shape[0]

    def qcol(cidx):
        v = pqb[:, cidx].reshape(_BQ2, 1, 1)
        return jnp.broadcast_to(v, (_BQ2, K, 1)).reshape(eb, 1)

    d0 = qcol(0) - pg[:, 0:1]
    d1 = qcol(1) - pg[:, 1:2]
    d2c = qcol(2) - pg[:, 2:3]
    d2b = (d0 * d0 + d1 * d1) + d2c * d2c
    dist = jnp.sqrt(d2b)
    delta = dist - off_ref[...]
    g = jnp.exp(_COEFF * delta * delta)
    t = jnp.dot(g, dn1_ref[...], preferred_element_type=jnp.float32)
    t = t + dn1b_ref[...]
    t = jnp.log(1.0 + jnp.exp(t)) - _LOG2
    w = jnp.dot(t, dn2_ref[...], preferred_element_type=jnp.float32)
    w = w + dn2b_ref[...]
    c = 0.5 * (jnp.cos(dist * (PI / CUTOFF)) + 1.0)
    c = jnp.where(dist <= CUTOFF, c, 0.0)
    p = (w * c) * g_ref[...]
    y0 = jnp.sum(p.reshape(_BQ2, K, NF), axis=1)
    csum = jnp.sum(c.reshape(_BQ2, K, 1), axis=1)
    y = jnp.dot(y0, lin2_ref[...], preferred_element_type=jnp.float32)
    y = y + csum * lin2b_ref[...]
    h = jnp.dot(y, c1_ref[...], preferred_element_type=jnp.float32)
    h = h + c1b_ref[...]
    h = jnp.log(1.0 + jnp.exp(h)) - _LOG2
    out_ref[...] = jnp.dot(h, c2_ref[...],
                           preferred_element_type=jnp.float32) + c2b_ref[...]


def _combine(pos_query, g_rows, pos_g, offset, dn1_W, dn1_b, dn2_W, dn2_b,
             lin2_W, lin2_b, cls1_W, cls1_b, cls2_W, cls2_b):
    e = g_rows.shape[0]
    nq = e // K
    o = cls2_W.shape[1]
    grid = nq // _BQ2
    eb = _BQ2 * K
    full = lambda a: pl.BlockSpec(a.shape, lambda i: tuple(0 for _ in a.shape))
    return pl.pallas_call(
        _combine_body,
        grid=(grid,),
        in_specs=[
            pl.BlockSpec((_BQ2, 3), lambda i: (i, 0)),
            pl.BlockSpec((eb, NF), lambda i: (i, 0)),
            pl.BlockSpec((eb, NP3), lambda i: (i, 0)),
            full(offset), full(dn1_W), full(dn1_b), full(dn2_W), full(dn2_b),
            full(lin2_W), full(lin2_b), full(cls1_W), full(cls1_b),
            full(cls2_W), full(cls2_b),
        ],
        out_specs=pl.BlockSpec((_BQ2, o), lambda i: (i, 0)),
        out_shape=jax.ShapeDtypeStruct((nq, o), jnp.float32),
    )(pos_query, g_rows, pos_g, offset, dn1_W, dn1_b, dn2_W, dn2_b, lin2_W,
      lin2_b, cls1_W, cls1_b, cls2_W, cls2_b)


def kernel(pos_query, pos_ctx, node_attr_ctx, lin1_W, lin2_W, lin2_b,
           dn1_W, dn1_b, dn2_W, dn2_b, cls1_W, cls1_b, cls2_W, cls2_b):
    pos_ctx_T = pos_ctx.T
    pos_pad = jnp.pad(pos_ctx, ((0, 0), (0, NP3 - 3)))
    x1 = _x1(node_attr_ctx, lin1_W)
    nbr_t = _knn(pos_query, pos_ctx_T)
    col = nbr_t.transpose(0, 2, 1).reshape(-1)
    g_rows, pos_g = _gather(x1, pos_pad, col)
    offset = jnp.asarray(_OFFSET_NP).reshape(1, NF)
    return _combine(
        pos_query, g_rows, pos_g, offset, dn1_W, dn1_b.reshape(1, NF), dn2_W,
        dn2_b.reshape(1, NF), lin2_W, lin2_b.reshape(1, NF), cls1_W,
        cls1_b.reshape(1, NF), cls2_W, cls2_b.reshape(1, -1))
